# baseline (speedup 1.0000x reference)
import dataclasses
import functools

import jax
import jax.numpy as jnp
from jax import lax
from jax.experimental import pallas as pl
from jax.experimental.pallas import tpu as pltpu
from jax.experimental.pallas import tpu_sc as plsc

N_NODES = 10000
D_FEAT = 128
N_EDGES = 320000

NC = 2
NS = 16
NW = NC * NS
E_TILE = N_EDGES // NW
WIN = 50
NWIN = E_TILE // WIN
ROWS_SUB = 624
ROWS_TAIL = N_NODES - NS * ROWS_SUB
LANES = 16
NBUF = 4
NIDX = 4


def _spmm_partials(src, rows3d, cols3d, vals, zeros):
    mesh = plsc.VectorSubcoreMesh(core_axis_name="c", subcore_axis_name="s")
    cp = pltpu.CompilerParams()
    if "needs_layout_passes" in pltpu.CompilerParams.__dataclass_fields__:
        cp = dataclasses.replace(cp, needs_layout_passes=False)

    @functools.partial(
        pl.kernel,
        compiler_params=cp,
        out_type=jax.ShapeDtypeStruct((NC, N_NODES, D_FEAT), jnp.float32),
        mesh=mesh,
        scratch_types=[
            pltpu.VMEM((E_TILE,), jnp.float32),
            pltpu.VMEM_SHARED((N_NODES, D_FEAT), jnp.float32),
            pltpu.SemaphoreType.DMA,
        ] + [pltpu.VMEM((1, WIN), jnp.int32)] * (2 * NIDX)
          + [pltpu.VMEM((WIN, D_FEAT), jnp.float32)] * NBUF
          + [pltpu.SemaphoreType.DMA] * (2 * NIDX + 2 * NBUF),
    )
    def k(src_hbm, rows_hbm, cols_hbm, vals_hbm, zeros_hbm, out_hbm,
          valv, acc, sem, *rest):
        colw = rest[0:NIDX]
        roww = rest[NIDX:2 * NIDX]
        gring = rest[2 * NIDX:2 * NIDX + NBUF]
        csem = rest[2 * NIDX + NBUF:3 * NIDX + NBUF]
        rsem = rest[3 * NIDX + NBUF:4 * NIDX + NBUF]
        gsem = rest[4 * NIDX + NBUF:4 * NIDX + 2 * NBUF]
        ssem = rest[4 * NIDX + 2 * NBUF:]
        c = lax.axis_index("c")
        s = lax.axis_index("s")
        wid = c * NS + s
        wbase = wid * NWIN

        pltpu.sync_copy(vals_hbm.at[pl.ds(wid * E_TILE, E_TILE)], valv)

        pltpu.sync_copy(zeros_hbm.at[pl.ds(s * ROWS_SUB, ROWS_SUB)],
                        acc.at[pl.ds(s * ROWS_SUB, ROWS_SUB)])

        @pl.when(s == 0)
        def _():
            pltpu.sync_copy(zeros_hbm.at[pl.ds(NS * ROWS_SUB, ROWS_TAIL)],
                            acc.at[pl.ds(NS * ROWS_SUB, ROWS_TAIL)])

        plsc.subcore_barrier()

        def issue_cols(w, i):
            pltpu.async_copy(cols_hbm.at[wbase + w], colw[i], csem[i])

        def issue_rows(w, i):
            pltpu.async_copy(rows_hbm.at[wbase + w], roww[i], rsem[i])

        def wait_idx_cols(w, i):
            pltpu.make_async_copy(cols_hbm.at[wbase + w], colw[i],
                                  csem[i]).wait()

        def wait_idx_rows(w, i):
            pltpu.make_async_copy(rows_hbm.at[wbase + w], roww[i],
                                  rsem[i]).wait()

        def issue_gather(w, i, b):
            pltpu.async_copy(src_hbm.at[colw[i].at[0]], gring[b], gsem[b])

        def wait_gather(w, i, b):
            pltpu.make_async_copy(src_hbm.at[colw[i].at[0]], gring[b],
                                  gsem[b]).wait()

        def issue_scatter(w, i, b):
            pltpu.async_copy(gring[b], acc.at[roww[i].at[0]], ssem[b],
                             add=True)

        def wait_scatter(w, i, b):
            pltpu.make_async_copy(gring[b], acc.at[roww[i].at[0]],
                                  ssem[b]).wait()

        for w in range(NIDX):
            issue_cols(w, w)
            issue_rows(w, w)
        for w in range(2):
            wait_idx_cols(w, w)
            issue_gather(w, w, w)

        @pl.loop(0, NWIN, step=NIDX)
        def _(w0):
            for i in range(NIDX):
                w = w0 + i
                b = i
                wait_gather(w, i, b)

                @pl.when(w + NIDX < NWIN)
                def _():
                    issue_cols(w + NIDX, i)

                gbuf = gring[b]

                @plsc.parallel_loop(0, WIN, unroll=10)
                def _(e):
                    idx16 = jnp.full((LANES,), w * WIN + e, jnp.int32)
                    vbc = plsc.load_gather(valv, [idx16])
                    for j in range(D_FEAT // LANES):
                        sl = (e, pl.ds(j * LANES, LANES))
                        gbuf[sl] = gbuf[sl] * vbc

                wait_idx_rows(w, i)
                issue_scatter(w, i, b)

                i2 = (i + 2) % NIDX

                @pl.when(w >= 2)
                def _():
                    wait_scatter(w - 2, i2, i2)

                @pl.when(jnp.logical_and(w >= 2, w + 2 < NWIN))
                def _():
                    issue_rows(w + 2, i2)

                @pl.when(w + 2 < NWIN)
                def _():
                    wait_idx_cols(w + 2, i2)
                    issue_gather(w + 2, i2, i2)

        wait_scatter(NWIN - 2, (NWIN - 2) % NIDX, (NWIN - 2) % NBUF)
        wait_scatter(NWIN - 1, (NWIN - 1) % NIDX, (NWIN - 1) % NBUF)

        plsc.subcore_barrier()
        pltpu.sync_copy(acc.at[pl.ds(s * ROWS_SUB, ROWS_SUB)],
                        out_hbm.at[c].at[pl.ds(s * ROWS_SUB, ROWS_SUB)])

        @pl.when(s == 0)
        def _():
            pltpu.sync_copy(acc.at[pl.ds(NS * ROWS_SUB, ROWS_TAIL)],
                            out_hbm.at[c].at[pl.ds(NS * ROWS_SUB, ROWS_TAIL)])

    return k(src, rows3d, cols3d, vals, zeros)


def _combine_scaled(p0, p1, alph):
    def body(a_ref, p0_ref, p1_ref, o_ref):
        o_ref[...] = a_ref[0, 0] * (p0_ref[...] + p1_ref[...])

    return pl.pallas_call(
        body,
        out_shape=jax.ShapeDtypeStruct((N_NODES, D_FEAT), jnp.float32),
        in_specs=[
            pl.BlockSpec(memory_space=pltpu.SMEM),
            pl.BlockSpec(),
            pl.BlockSpec(),
        ],
        out_specs=pl.BlockSpec(),
    )(alph, p0, p1)


def _finalize(q0, q1, x):
    def body(q0_ref, q1_ref, x_ref, o_ref):
        xc = jnp.nan_to_num(x_ref[...], nan=0.0, posinf=1e6, neginf=-1e6)
        o_ref[...] = jnp.clip((q0_ref[...] + q1_ref[...]) - xc, -5.0, 5.0)

    return pl.pallas_call(
        body,
        out_shape=jax.ShapeDtypeStruct((N_NODES, D_FEAT), jnp.float32),
    )(q0, q1, x)


def kernel(t, x, rows, cols, vals, alpha_train, temperature):
    del t
    rows3d = rows.astype(jnp.int32).reshape(N_EDGES // WIN, 1, WIN)
    cols3d = cols.astype(jnp.int32).reshape(N_EDGES // WIN, 1, WIN)
    vals = vals.astype(jnp.float32)
    zeros = jnp.zeros((N_NODES, D_FEAT), jnp.float32)
    alph = jax.nn.sigmoid(alpha_train * temperature).reshape(1, 1)

    p = _spmm_partials(x, rows3d, cols3d, vals, zeros)
    ax = _combine_scaled(p[0], p[1], alph)
    q = _spmm_partials(ax, rows3d, cols3d, vals, zeros)
    return _finalize(q[0], q[1], x)


# speedup vs baseline: 1.0023x; 1.0023x over previous
import dataclasses
import functools

import jax
import jax.numpy as jnp
from jax import lax
from jax.experimental import pallas as pl
from jax.experimental.pallas import tpu as pltpu
from jax.experimental.pallas import tpu_sc as plsc

N_NODES = 10000
D_FEAT = 128
N_EDGES = 320000

NC = 2
NS = 16
NW = NC * NS
E_TILE = N_EDGES // NW
WIN = 50
NWIN = E_TILE // WIN
ROWS_SUB = 624
ROWS_TAIL = N_NODES - NS * ROWS_SUB
LANES = 16
NBUF = 4
NIDX = 4


def _spmm_partials(src, rows3d, cols3d, vals, zeros):
    mesh = plsc.VectorSubcoreMesh(core_axis_name="c", subcore_axis_name="s")
    cp = pltpu.CompilerParams()
    if "needs_layout_passes" in pltpu.CompilerParams.__dataclass_fields__:
        cp = dataclasses.replace(cp, needs_layout_passes=False)

    @functools.partial(
        pl.kernel,
        compiler_params=cp,
        out_type=jax.ShapeDtypeStruct((NC, N_NODES, D_FEAT), jnp.float32),
        mesh=mesh,
        scratch_types=[
            pltpu.VMEM((E_TILE + LANES,), jnp.float32),
            pltpu.VMEM_SHARED((N_NODES, D_FEAT), jnp.float32),
            pltpu.SemaphoreType.DMA,
        ] + [pltpu.VMEM((1, WIN), jnp.int32)] * (2 * NIDX)
          + [pltpu.VMEM((WIN, D_FEAT), jnp.float32)] * NBUF
          + [pltpu.SemaphoreType.DMA] * (2 * NIDX + 2 * NBUF),
    )
    def k(src_hbm, rows_hbm, cols_hbm, vals_hbm, zeros_hbm, out_hbm,
          valv, acc, sem, *rest):
        colw = rest[0:NIDX]
        roww = rest[NIDX:2 * NIDX]
        gring = rest[2 * NIDX:2 * NIDX + NBUF]
        csem = rest[2 * NIDX + NBUF:3 * NIDX + NBUF]
        rsem = rest[3 * NIDX + NBUF:4 * NIDX + NBUF]
        gsem = rest[4 * NIDX + NBUF:4 * NIDX + 2 * NBUF]
        ssem = rest[4 * NIDX + 2 * NBUF:]
        c = lax.axis_index("c")
        s = lax.axis_index("s")
        wid = c * NS + s
        wbase = wid * NWIN

        pltpu.sync_copy(vals_hbm.at[pl.ds(wid * E_TILE, E_TILE)],
                        valv.at[pl.ds(0, E_TILE)])

        pltpu.sync_copy(zeros_hbm.at[pl.ds(s * ROWS_SUB, ROWS_SUB)],
                        acc.at[pl.ds(s * ROWS_SUB, ROWS_SUB)])

        @pl.when(s == 0)
        def _():
            pltpu.sync_copy(zeros_hbm.at[pl.ds(NS * ROWS_SUB, ROWS_TAIL)],
                            acc.at[pl.ds(NS * ROWS_SUB, ROWS_TAIL)])

        plsc.subcore_barrier()

        def issue_cols(w, i):
            pltpu.async_copy(cols_hbm.at[wbase + w], colw[i], csem[i])

        def issue_rows(w, i):
            pltpu.async_copy(rows_hbm.at[wbase + w], roww[i], rsem[i])

        def wait_idx_cols(w, i):
            pltpu.make_async_copy(cols_hbm.at[wbase + w], colw[i],
                                  csem[i]).wait()

        def wait_idx_rows(w, i):
            pltpu.make_async_copy(rows_hbm.at[wbase + w], roww[i],
                                  rsem[i]).wait()

        def issue_gather(w, i, b):
            pltpu.async_copy(src_hbm.at[colw[i].at[0]], gring[b], gsem[b])

        def wait_gather(w, i, b):
            pltpu.make_async_copy(src_hbm.at[colw[i].at[0]], gring[b],
                                  gsem[b]).wait()

        def issue_scatter(w, i, b):
            pltpu.async_copy(gring[b], acc.at[roww[i].at[0]], ssem[b],
                             add=True)

        def wait_scatter(w, i, b):
            pltpu.make_async_copy(gring[b], acc.at[roww[i].at[0]],
                                  ssem[b]).wait()

        for w in range(NIDX):
            issue_cols(w, w)
            issue_rows(w, w)
        for w in range(2):
            wait_idx_cols(w, w)
            issue_gather(w, w, w)

        @pl.loop(0, NWIN, step=NIDX)
        def _(w0):
            for i in range(NIDX):
                w = w0 + i
                b = i
                wait_gather(w, i, b)

                @pl.when(w + NIDX < NWIN)
                def _():
                    issue_cols(w + NIDX, i)

                gbuf = gring[b]

                @plsc.parallel_loop(0, WIN // 10, unroll=2)
                def _(g):
                    e0 = g * 10
                    vv = valv[pl.ds(w * WIN + e0, LANES)]
                    for u in range(10):
                        vbc = jnp.broadcast_to(vv[u], (LANES,))
                        for j in range(D_FEAT // LANES):
                            sl = (e0 + u, pl.ds(j * LANES, LANES))
                            gbuf[sl] = gbuf[sl] * vbc

                wait_idx_rows(w, i)
                issue_scatter(w, i, b)

                i2 = (i + 2) % NIDX

                @pl.when(w >= 2)
                def _():
                    wait_scatter(w - 2, i2, i2)

                @pl.when(jnp.logical_and(w >= 2, w + 2 < NWIN))
                def _():
                    issue_rows(w + 2, i2)

                @pl.when(w + 2 < NWIN)
                def _():
                    wait_idx_cols(w + 2, i2)
                    issue_gather(w + 2, i2, i2)

        wait_scatter(NWIN - 2, (NWIN - 2) % NIDX, (NWIN - 2) % NBUF)
        wait_scatter(NWIN - 1, (NWIN - 1) % NIDX, (NWIN - 1) % NBUF)

        plsc.subcore_barrier()
        pltpu.sync_copy(acc.at[pl.ds(s * ROWS_SUB, ROWS_SUB)],
                        out_hbm.at[c].at[pl.ds(s * ROWS_SUB, ROWS_SUB)])

        @pl.when(s == 0)
        def _():
            pltpu.sync_copy(acc.at[pl.ds(NS * ROWS_SUB, ROWS_TAIL)],
                            out_hbm.at[c].at[pl.ds(NS * ROWS_SUB, ROWS_TAIL)])

    return k(src, rows3d, cols3d, vals, zeros)


def _combine_scaled(p0, p1, alph):
    def body(a_ref, p0_ref, p1_ref, o_ref):
        o_ref[...] = a_ref[0, 0] * (p0_ref[...] + p1_ref[...])

    return pl.pallas_call(
        body,
        out_shape=jax.ShapeDtypeStruct((N_NODES, D_FEAT), jnp.float32),
        in_specs=[
            pl.BlockSpec(memory_space=pltpu.SMEM),
            pl.BlockSpec(),
            pl.BlockSpec(),
        ],
        out_specs=pl.BlockSpec(),
    )(alph, p0, p1)


def _finalize(q0, q1, x):
    def body(q0_ref, q1_ref, x_ref, o_ref):
        xc = jnp.nan_to_num(x_ref[...], nan=0.0, posinf=1e6, neginf=-1e6)
        o_ref[...] = jnp.clip((q0_ref[...] + q1_ref[...]) - xc, -5.0, 5.0)

    return pl.pallas_call(
        body,
        out_shape=jax.ShapeDtypeStruct((N_NODES, D_FEAT), jnp.float32),
    )(q0, q1, x)


def kernel(t, x, rows, cols, vals, alpha_train, temperature):
    del t
    rows3d = rows.astype(jnp.int32).reshape(N_EDGES // WIN, 1, WIN)
    cols3d = cols.astype(jnp.int32).reshape(N_EDGES // WIN, 1, WIN)
    vals = vals.astype(jnp.float32)
    zeros = jnp.zeros((N_NODES, D_FEAT), jnp.float32)
    alph = jax.nn.sigmoid(alpha_train * temperature).reshape(1, 1)

    p = _spmm_partials(x, rows3d, cols3d, vals, zeros)
    ax = _combine_scaled(p[0], p[1], alph)
    q = _spmm_partials(ax, rows3d, cols3d, vals, zeros)
    return _finalize(q[0], q[1], x)
